# Initial kernel scaffold; baseline (speedup 1.0000x reference)
import jax
import jax.numpy as jnp
from jax.experimental import pallas as pl


def kernel(x, edge_index, edge_weight, W1, b1, g1, beta1, W2, b2, g2, beta2):
    raise NotImplementedError("write your pallas kernel here")


# speedup vs baseline: 5.1686x; 5.1686x over previous
import functools

import jax
import jax.numpy as jnp
from jax import lax
from jax.experimental import pallas as pl
from jax.experimental.pallas import tpu as pltpu
from jax.experimental.pallas import tpu_sc as plsc

N = 10000
D = 256
E = 160000
L = 128
BN_EPS = 1e-5

NUM_CORES = 2
NUM_TILES = 16
CHUNK = 128
N_PAD = 10240
ROWS_PT = N_PAD // NUM_TILES
E_PAD = 163840
CHUNKS_TOTAL = E_PAD // CHUNK
CPT = CHUNKS_TOTAL // NUM_TILES


def _sc_agg_body(x2_hbm, xs_hbm, srcs_hbm, dsts_hbm, out_hbm,
                 src_v, dst_v, gbuf, sem, agg_sh):
    c = lax.axis_index("c")
    s = lax.axis_index("s")
    r0 = s * ROWS_PT
    pltpu.sync_copy(xs_hbm.at[c, pl.ds(r0, ROWS_PT)],
                    agg_sh.at[pl.ds(r0, ROWS_PT)])
    base = s * CPT
    pltpu.sync_copy(srcs_hbm.at[c, pl.ds(base, CPT)], src_v)
    pltpu.sync_copy(dsts_hbm.at[pl.ds(base, CPT)], dst_v)
    plsc.subcore_barrier()

    def step(j, carry):
        pltpu.async_copy(x2_hbm.at[src_v.at[j]], gbuf, sem).wait()
        pltpu.sync_copy(gbuf, agg_sh.at[dst_v.at[j]], add=True)
        return carry

    lax.fori_loop(0, CPT, step, 0)
    plsc.subcore_barrier()
    full = 632
    rem = N - (NUM_TILES - 1) * full

    @pl.when(s < NUM_TILES - 1)
    def _copy_full():
        pltpu.sync_copy(agg_sh.at[pl.ds(s * full, full)],
                        out_hbm.at[c, pl.ds(s * full, full)])

    @pl.when(s == NUM_TILES - 1)
    def _copy_rem():
        pltpu.sync_copy(agg_sh.at[pl.ds((NUM_TILES - 1) * full, rem)],
                        out_hbm.at[c, pl.ds((NUM_TILES - 1) * full, rem)])


@functools.cache
def _sc_agg():
    return pl.kernel(
        _sc_agg_body,
        out_type=jax.ShapeDtypeStruct((NUM_CORES, N, L), jnp.float32),
        mesh=plsc.VectorSubcoreMesh(core_axis_name="c", subcore_axis_name="s",
                                    num_cores=NUM_CORES,
                                    num_subcores=NUM_TILES),
        scratch_types=[
            pltpu.VMEM((CPT, CHUNK), jnp.int32),
            pltpu.VMEM((CPT, CHUNK), jnp.int32),
            pltpu.VMEM((CHUNK, L), jnp.float32),
            pltpu.SemaphoreType.DMA,
            pltpu.VMEM_SHARED((N_PAD, L), jnp.float32),
        ],
    )


def _tc_mlp_body(aggs_ref, w1_ref, b1_ref, g1_ref, bt1_ref,
                 w2_ref, b2_ref, g2_ref, bt2_ref, out_ref):
    w1 = w1_ref[...]
    h = (lax.dot_general(aggs_ref[0], w1[:, :L], (((1,), (1,)), ((), ())))
         + lax.dot_general(aggs_ref[1], w1[:, L:], (((1,), (1,)), ((), ()))))
    h = h + b1_ref[...]
    mean = jnp.mean(h, axis=0, keepdims=True)
    d = h - mean
    var = jnp.mean(d * d, axis=0, keepdims=True)
    h = d * lax.rsqrt(var + BN_EPS) * g1_ref[...] + bt1_ref[...]
    h = lax.dot_general(h, w2_ref[...], (((1,), (1,)), ((), ())))
    h = h + b2_ref[...]
    mean = jnp.mean(h, axis=0, keepdims=True)
    d = h - mean
    var = jnp.mean(d * d, axis=0, keepdims=True)
    out_ref[...] = d * lax.rsqrt(var + BN_EPS) * g2_ref[...] + bt2_ref[...]


_tc_mlp = pl.pallas_call(
    _tc_mlp_body,
    out_shape=jax.ShapeDtypeStruct((N, D), jnp.float32),
)


def kernel(x, edge_index, edge_weight, W1, b1, g1, beta1, W2, b2, g2, beta2):
    del edge_weight
    src = edge_index[0]
    dst = edge_index[1]
    pad_e = E_PAD - E
    pad_src = jnp.arange(pad_e, dtype=jnp.int32) * 17 % N
    src_p = jnp.concatenate([src, pad_src])
    junk = (N + (jnp.arange(pad_e, dtype=jnp.int32) % (N_PAD - N)))
    dst_p = jnp.concatenate([dst, junk])
    src2 = 2 * src_p
    srcs = jnp.stack([src2, src2 + 1]).reshape(2, CHUNKS_TOTAL, CHUNK)
    dsts = dst_p.reshape(CHUNKS_TOTAL, CHUNK)
    x2 = x.reshape(2 * N, L)
    xs = jnp.pad(x.reshape(N, 2, L).transpose(1, 0, 2),
                 ((0, 0), (0, N_PAD - N), (0, 0)))

    aggs = _sc_agg()(x2, xs, srcs, dsts)

    return _tc_mlp(aggs, W1, b1[None], g1[None], beta1[None],
                   W2, b2[None], g2[None], beta2[None])
